# baseline (speedup 1.0000x reference)
import functools

import jax
import jax.numpy as jnp
from jax import lax
from jax.experimental import pallas as pl
from jax.experimental.pallas import tpu as pltpu
from jax.experimental.pallas import tpu_sc as plsc

NUM_EMBEDDINGS = 100000
PRETRAIN_DIM = 300
_DPAD = 304
BATCH = 4096
HIST_LEN = 50

_info = plsc.get_sparse_core_info()
_NC, _NS = _info.num_cores, _info.num_subcores
_NW = _NC * _NS

_B = BATCH * HIST_LEN
_BPW = _B // _NW
_CHUNK = 128
_NCHUNK = _BPW // _CHUNK


def _make_kernel():
    mesh = plsc.VectorSubcoreMesh(core_axis_name="c", subcore_axis_name="s")

    @functools.partial(
        pl.kernel,
        mesh=mesh,
        compiler_params=pltpu.CompilerParams(use_tc_tiling_on_sc=False),
        out_type=jax.ShapeDtypeStruct((_B, _DPAD), jnp.float32),
        scratch_types=[
            pltpu.VMEM((_NCHUNK, _CHUNK), jnp.int32),
            pltpu.VMEM((_CHUNK, _DPAD), jnp.float32),
            pltpu.SemaphoreType.DMA,
        ],
    )
    def gather_kernel(table_hbm, idx_hbm, out_hbm, idx_v, rows, g_sem):
        wid = lax.axis_index("s") * _NC + lax.axis_index("c")
        base = wid * _BPW
        pltpu.sync_copy(idx_hbm.at[wid], idx_v)

        def loop_body(j, carry):
            pltpu.async_copy(table_hbm.at[idx_v.at[j]], rows, g_sem).wait()
            pltpu.sync_copy(rows, out_hbm.at[pl.ds(base + j * _CHUNK, _CHUNK)])
            return carry

        lax.fori_loop(0, _NCHUNK, loop_body, 0)

    return gather_kernel


_kernel = _make_kernel()


def kernel(x, table):
    idx = x.reshape(_NW, _NCHUNK, _CHUNK).astype(jnp.int32)
    table_p = jnp.pad(table, ((0, 0), (0, _DPAD - PRETRAIN_DIM)))
    out = _kernel(table_p, idx)
    return out[:, :PRETRAIN_DIM].reshape(BATCH, HIST_LEN, PRETRAIN_DIM)


# speedup vs baseline: 1.3560x; 1.3560x over previous
import functools

import jax
import jax.numpy as jnp
from jax import lax
from jax.experimental import pallas as pl
from jax.experimental.pallas import tpu as pltpu
from jax.experimental.pallas import tpu_sc as plsc

NUM_EMBEDDINGS = 100000
PRETRAIN_DIM = 300
_DPAD = 384
BATCH = 4096
HIST_LEN = 50

_info = plsc.get_sparse_core_info()
_NC, _NS = _info.num_cores, _info.num_subcores
_NW = _NC * _NS

_B = BATCH * HIST_LEN
_BPW = _B // _NW
_CHUNK = 128
_NCHUNK = _BPW // _CHUNK


def _make_kernel():
    mesh = plsc.VectorSubcoreMesh(core_axis_name="c", subcore_axis_name="s")

    @functools.partial(
        pl.kernel,
        mesh=mesh,
        out_type=jax.ShapeDtypeStruct((_B, _DPAD), jnp.float32),
        scratch_types=[
            pltpu.VMEM((_NCHUNK, _CHUNK), jnp.int32),
            pltpu.VMEM((_CHUNK, _DPAD), jnp.float32),
            pltpu.SemaphoreType.DMA,
        ],
    )
    def gather_kernel(table_hbm, idx_hbm, out_hbm, idx_v, rows, g_sem):
        wid = lax.axis_index("s") * _NC + lax.axis_index("c")
        base = wid * _BPW
        pltpu.sync_copy(idx_hbm.at[wid], idx_v)

        def loop_body(j, carry):
            pltpu.async_copy(table_hbm.at[idx_v.at[j]], rows, g_sem).wait()
            pltpu.sync_copy(rows, out_hbm.at[pl.ds(base + j * _CHUNK, _CHUNK)])
            return carry

        lax.fori_loop(0, _NCHUNK, loop_body, 0)

    return gather_kernel


_kernel = _make_kernel()


def kernel(x, table):
    idx = x.reshape(_NW, _NCHUNK, _CHUNK).astype(jnp.int32)
    table_p = jnp.pad(table, ((0, 0), (0, _DPAD - PRETRAIN_DIM)))
    out = _kernel(table_p, idx)
    return out[:, :PRETRAIN_DIM].reshape(BATCH, HIST_LEN, PRETRAIN_DIM)


# speedup vs baseline: 1.7004x; 1.2540x over previous
import functools

import jax
import jax.numpy as jnp
from jax import lax
from jax.experimental import pallas as pl
from jax.experimental.pallas import tpu as pltpu
from jax.experimental.pallas import tpu_sc as plsc

NUM_EMBEDDINGS = 100000
PRETRAIN_DIM = 300
_DPAD = 384
BATCH = 4096
HIST_LEN = 50

_info = plsc.get_sparse_core_info()
_NC, _NS = _info.num_cores, _info.num_subcores
_NW = _NC * _NS

_B = BATCH * HIST_LEN
_BPW = _B // _NW
_CHUNK = 128
_NCHUNK = _BPW // _CHUNK


def _make_gather():
    mesh = plsc.VectorSubcoreMesh(core_axis_name="c", subcore_axis_name="s")

    @functools.partial(
        pl.kernel,
        mesh=mesh,
        out_type=jax.ShapeDtypeStruct((_B, _DPAD), jnp.float32),
        scratch_types=[
            pltpu.VMEM((_NCHUNK, _CHUNK), jnp.int32),
            pltpu.VMEM((_CHUNK, _DPAD), jnp.float32),
            pltpu.SemaphoreType.DMA,
        ],
    )
    def gather_kernel(table_hbm, idx_hbm, out_hbm, idx_v, rows, g_sem):
        wid = lax.axis_index("s") * _NC + lax.axis_index("c")
        base = wid * _BPW
        pltpu.sync_copy(idx_hbm.at[wid], idx_v)

        def loop_body(j, carry):
            pltpu.async_copy(table_hbm.at[idx_v.at[j]], rows, g_sem).wait()
            pltpu.sync_copy(rows, out_hbm.at[pl.ds(base + j * _CHUNK, _CHUNK)])
            return carry

        lax.fori_loop(0, _NCHUNK, loop_body, 0)

    return gather_kernel


_PAD_ROWS = 2000


def _pad_body(t_ref, o_ref):
    o_ref[:, :PRETRAIN_DIM] = t_ref[...]
    o_ref[:, PRETRAIN_DIM:] = jnp.zeros(
        (_PAD_ROWS, _DPAD - PRETRAIN_DIM), jnp.float32
    )


_pad_table = pl.pallas_call(
    _pad_body,
    grid=(NUM_EMBEDDINGS // _PAD_ROWS,),
    in_specs=[pl.BlockSpec((_PAD_ROWS, PRETRAIN_DIM), lambda i: (i, 0))],
    out_specs=pl.BlockSpec((_PAD_ROWS, _DPAD), lambda i: (i, 0)),
    out_shape=jax.ShapeDtypeStruct((NUM_EMBEDDINGS, _DPAD), jnp.float32),
)


_DB = 16


def _depad_body(g_ref, o_ref):
    o_ref[...] = g_ref[:, :PRETRAIN_DIM].reshape(_DB, HIST_LEN, PRETRAIN_DIM)


_depad = pl.pallas_call(
    _depad_body,
    grid=(BATCH // _DB,),
    in_specs=[pl.BlockSpec((_DB * HIST_LEN, _DPAD), lambda i: (i, 0))],
    out_specs=pl.BlockSpec((_DB, HIST_LEN, PRETRAIN_DIM), lambda i: (i, 0, 0)),
    out_shape=jax.ShapeDtypeStruct((BATCH, HIST_LEN, PRETRAIN_DIM), jnp.float32),
)


_gather = _make_gather()


def kernel(x, table):
    idx = x.reshape(_NW, _NCHUNK, _CHUNK).astype(jnp.int32)
    table_p = _pad_table(table)
    out = _gather(table_p, idx)
    return _depad(out)


# speedup vs baseline: 1.7931x; 1.0545x over previous
import functools

import jax
import jax.numpy as jnp
from jax import lax
from jax.experimental import pallas as pl
from jax.experimental.pallas import tpu as pltpu
from jax.experimental.pallas import tpu_sc as plsc

NUM_EMBEDDINGS = 100000
PRETRAIN_DIM = 300
_DPAD = 384
BATCH = 4096
HIST_LEN = 50

_info = plsc.get_sparse_core_info()
_NC, _NS = _info.num_cores, _info.num_subcores
_NW = _NC * _NS

_B = BATCH * HIST_LEN
_S = 4
_R = _B // _S
_BPW = _R // _NW
_CHUNK = 64
_NCHUNK = _BPW // _CHUNK


def _make_gather():
    mesh = plsc.VectorSubcoreMesh(core_axis_name="c", subcore_axis_name="s")

    @functools.partial(
        pl.kernel,
        mesh=mesh,
        out_type=jax.ShapeDtypeStruct((_R, _DPAD), jnp.float32),
        scratch_types=[
            pltpu.VMEM((_NCHUNK, _CHUNK), jnp.int32),
            pltpu.VMEM((_CHUNK, _DPAD), jnp.float32),
            pltpu.SemaphoreType.DMA,
        ],
    )
    def gather_kernel(table_hbm, idx_hbm, out_hbm, idx_v, rows, g_sem):
        wid = lax.axis_index("s") * _NC + lax.axis_index("c")
        base = wid * _BPW
        pltpu.sync_copy(idx_hbm.at[wid], idx_v)

        def loop_body(j, carry):
            pltpu.async_copy(table_hbm.at[idx_v.at[j]], rows, g_sem).wait()
            pltpu.sync_copy(rows, out_hbm.at[pl.ds(base + j * _CHUNK, _CHUNK)])
            return carry

        lax.fori_loop(0, _NCHUNK, loop_body, 0)

    return gather_kernel


_gather = _make_gather()


_PAD_ROWS = 2000


def _pad_body(t_ref, o_ref):
    o_ref[:, :PRETRAIN_DIM] = t_ref[...]
    o_ref[:, PRETRAIN_DIM:] = jnp.zeros(
        (_PAD_ROWS, _DPAD - PRETRAIN_DIM), jnp.float32
    )


_pad_table = pl.pallas_call(
    _pad_body,
    grid=(NUM_EMBEDDINGS // _PAD_ROWS,),
    in_specs=[pl.BlockSpec((_PAD_ROWS, PRETRAIN_DIM), lambda i: (i, 0))],
    out_specs=pl.BlockSpec((_PAD_ROWS, _DPAD), lambda i: (i, 0)),
    out_shape=jax.ShapeDtypeStruct((NUM_EMBEDDINGS, _DPAD), jnp.float32),
)


_DB = 16
_BS = BATCH // _S
_OUT_SHAPE = jax.ShapeDtypeStruct((BATCH, HIST_LEN, PRETRAIN_DIM), jnp.float32)


def _depad_first_body(g_ref, o_ref):
    o_ref[...] = g_ref[:, :PRETRAIN_DIM].reshape(_DB, HIST_LEN, PRETRAIN_DIM)


def _depad_chain_body(g_ref, a_ref, o_ref):
    del a_ref
    o_ref[...] = g_ref[:, :PRETRAIN_DIM].reshape(_DB, HIST_LEN, PRETRAIN_DIM)


def _make_depad(s):
    bofs = s * _BS // _DB

    if s == 0:
        return pl.pallas_call(
            _depad_first_body,
            grid=(_BS // _DB,),
            in_specs=[pl.BlockSpec((_DB * HIST_LEN, _DPAD), lambda i: (i, 0))],
            out_specs=pl.BlockSpec(
                (_DB, HIST_LEN, PRETRAIN_DIM), lambda i: (bofs + i, 0, 0)
            ),
            out_shape=_OUT_SHAPE,
        )
    return pl.pallas_call(
        _depad_chain_body,
        grid=(_BS // _DB,),
        in_specs=[
            pl.BlockSpec((_DB * HIST_LEN, _DPAD), lambda i: (i, 0)),
            pl.BlockSpec((1, 8, 128), lambda i: (0, 0, 0)),
        ],
        out_specs=pl.BlockSpec(
            (_DB, HIST_LEN, PRETRAIN_DIM), lambda i: (bofs + i, 0, 0)
        ),
        out_shape=_OUT_SHAPE,
        input_output_aliases={1: 0},
    )


_depads = [_make_depad(s) for s in range(_S)]


def kernel(x, table):
    idx = x.reshape(_S, _NW, _NCHUNK, _CHUNK).astype(jnp.int32)
    table_p = _pad_table(table)
    gs = [_gather(table_p, idx[s]) for s in range(_S)]
    out = _depads[0](gs[0])
    for s in range(1, _S):
        out = _depads[s](gs[s], out)
    return out


# speedup vs baseline: 1.8423x; 1.0274x over previous
import functools

import jax
import jax.numpy as jnp
from jax import lax
from jax.experimental import pallas as pl
from jax.experimental.pallas import tpu as pltpu
from jax.experimental.pallas import tpu_sc as plsc

NUM_EMBEDDINGS = 100000
PRETRAIN_DIM = 300
_DPAD = 384
BATCH = 4096
HIST_LEN = 50

_info = plsc.get_sparse_core_info()
_NC, _NS = _info.num_cores, _info.num_subcores
_NW = _NC * _NS

_B = BATCH * HIST_LEN
_BPW = _B // _NW
_CHUNK = 128
_NCHUNK = _BPW // _CHUNK


def _make_gather():
    mesh = plsc.VectorSubcoreMesh(core_axis_name="c", subcore_axis_name="s")

    @functools.partial(
        pl.kernel,
        mesh=mesh,
        out_type=jax.ShapeDtypeStruct((_B, _DPAD), jnp.float32),
        scratch_types=[
            pltpu.VMEM((_NCHUNK, _CHUNK), jnp.int32),
            pltpu.VMEM((_CHUNK, _DPAD), jnp.float32),
            pltpu.SemaphoreType.DMA,
        ],
    )
    def gather_kernel(table_hbm, idx_hbm, out_hbm, idx_v, rows, g_sem):
        wid = lax.axis_index("s") * _NC + lax.axis_index("c")
        base = wid * _BPW
        pltpu.sync_copy(idx_hbm.at[wid], idx_v)

        def loop_body(j, carry):
            pltpu.async_copy(table_hbm.at[idx_v.at[j]], rows, g_sem).wait()
            pltpu.sync_copy(rows, out_hbm.at[pl.ds(base + j * _CHUNK, _CHUNK)])
            return carry

        lax.fori_loop(0, _NCHUNK, loop_body, 0)

    return gather_kernel


_gather = _make_gather()


_PNB = 4
_PAD_ROWS = 2000
_PAD_STEPS = NUM_EMBEDDINGS // _PAD_ROWS


def _pad_body(t_hbm, o_hbm, vin, vout, sin, sout):
    def in_dma(i, s):
        return pltpu.make_async_copy(
            t_hbm.at[pl.ds(i * _PAD_ROWS, _PAD_ROWS)], vin.at[s], sin.at[s]
        )

    def out_dma(i, s):
        return pltpu.make_async_copy(
            vout.at[s], o_hbm.at[pl.ds(i * _PAD_ROWS, _PAD_ROWS)], sout.at[s]
        )

    for j in range(_PNB):
        in_dma(j, j).start()

    def step(i, carry):
        s = lax.rem(i, _PNB)
        in_dma(i, s).wait()

        @pl.when(i >= _PNB)
        def _():
            out_dma(i - _PNB, s).wait()

        vout[s, :, :PRETRAIN_DIM] = vin[s]
        out_dma(i, s).start()

        @pl.when(i + _PNB < _PAD_STEPS)
        def _():
            in_dma(i + _PNB, s).start()

        return carry

    lax.fori_loop(0, _PAD_STEPS, step, 0)

    def drain(i, carry):
        out_dma(i, lax.rem(i, _PNB)).wait()
        return carry

    lax.fori_loop(_PAD_STEPS - _PNB, _PAD_STEPS, drain, 0)


_pad_table = pl.pallas_call(
    _pad_body,
    in_specs=[pl.BlockSpec(memory_space=pltpu.HBM)],
    out_specs=pl.BlockSpec(memory_space=pltpu.HBM),
    out_shape=jax.ShapeDtypeStruct((NUM_EMBEDDINGS, _DPAD), jnp.float32),
    scratch_shapes=[
        pltpu.VMEM((_PNB, _PAD_ROWS, PRETRAIN_DIM), jnp.float32),
        pltpu.VMEM((_PNB, _PAD_ROWS, _DPAD), jnp.float32),
        pltpu.SemaphoreType.DMA((_PNB,)),
        pltpu.SemaphoreType.DMA((_PNB,)),
    ],
)


_DNB = 6
_DB = 8
_DROWS = _DB * HIST_LEN
_DEPAD_STEPS = BATCH // _DB


def _depad_body(g_hbm, o_hbm, vin, vout, sin, sout):
    def in_dma(i, s):
        return pltpu.make_async_copy(
            g_hbm.at[pl.ds(i * _DROWS, _DROWS)], vin.at[s], sin.at[s]
        )

    def out_dma(i, s):
        return pltpu.make_async_copy(
            vout.at[s], o_hbm.at[pl.ds(i * _DB, _DB)], sout.at[s]
        )

    for j in range(_DNB):
        in_dma(j, j).start()

    def step(i, carry):
        s = lax.rem(i, _DNB)
        in_dma(i, s).wait()

        @pl.when(i >= _DNB)
        def _():
            out_dma(i - _DNB, s).wait()

        vout[s] = vin[s, :, :PRETRAIN_DIM].reshape(_DB, HIST_LEN, PRETRAIN_DIM)
        out_dma(i, s).start()

        @pl.when(i + _DNB < _DEPAD_STEPS)
        def _():
            in_dma(i + _DNB, s).start()

        return carry

    lax.fori_loop(0, _DEPAD_STEPS, step, 0)

    def drain(i, carry):
        out_dma(i, lax.rem(i, _DNB)).wait()
        return carry

    lax.fori_loop(_DEPAD_STEPS - _DNB, _DEPAD_STEPS, drain, 0)


_depad = pl.pallas_call(
    _depad_body,
    in_specs=[pl.BlockSpec(memory_space=pltpu.HBM)],
    out_specs=pl.BlockSpec(memory_space=pltpu.HBM),
    out_shape=jax.ShapeDtypeStruct((BATCH, HIST_LEN, PRETRAIN_DIM), jnp.float32),
    scratch_shapes=[
        pltpu.VMEM((_DNB, _DROWS, _DPAD), jnp.float32),
        pltpu.VMEM((_DNB, _DB, HIST_LEN, PRETRAIN_DIM), jnp.float32),
        pltpu.SemaphoreType.DMA((_DNB,)),
        pltpu.SemaphoreType.DMA((_DNB,)),
    ],
)


def kernel(x, table):
    idx = x.reshape(_NW, _NCHUNK, _CHUNK).astype(jnp.int32)
    table_p = _pad_table(table)
    out = _gather(table_p, idx)
    return _depad(out)
